# baseline (speedup 1.0000x reference)
import functools

import jax
import jax.numpy as jnp
from jax import lax
from jax.experimental import pallas as pl
from jax.experimental.pallas import tpu as pltpu
from jax.experimental.pallas import tpu_sc as plsc

VOCAB = 1000000
EMB = 64
MAXLEN = 200

NC = 2
NS = 16
NW = NC * NS
LANES = 16
NBUF = 4

TCW = 2048


def _sincos_position_encoding(max_length, embedding_dim, n=10000):
    k = jnp.arange(max_length, dtype=jnp.float32)[:, None]
    i = jnp.arange(embedding_dim // 2, dtype=jnp.float32)[None, :]
    denominator = jnp.power(float(n), 2.0 * i / embedding_dim)
    P = jnp.zeros((max_length, embedding_dim), dtype=jnp.float32)
    P = P.at[:, 0::2].set(jnp.sin(k / denominator))
    P = P.at[:, 1::2].set(jnp.cos(k / denominator))
    return P


def _tc_repack(table_t):
    nrows, ncols = table_t.shape

    def body(in_ref, out_ref):
        x = in_ref[...]
        eye = (lax.broadcasted_iota(jnp.int32, (EMB, EMB), 0)
               == lax.broadcasted_iota(jnp.int32, (EMB, EMB), 1)
               ).astype(jnp.float32)
        t = lax.dot_general(x, eye, (((0,), (0,)), ((), ())),
                            preferred_element_type=jnp.float32)
        t3 = t.reshape(TCW // 2, 2, EMB)
        out_ref[:, 0:EMB] = t3[:, 0, :]
        out_ref[:, EMB:2 * EMB] = t3[:, 1, :]

    grid = (ncols + TCW - 1) // TCW
    return pl.pallas_call(
        body,
        grid=(grid,),
        in_specs=[pl.BlockSpec((nrows, TCW), lambda i: (0, i))],
        out_specs=pl.BlockSpec((TCW // 2, 2 * EMB), lambda i: (i, 0)),
        out_shape=jax.ShapeDtypeStruct((ncols // 2, 2 * EMB), jnp.float32),
    )(table_t)


@functools.partial(jax.jit, static_argnames=("bsz", "seq"))
def _sc_lookup(xf, pf, table, penc, bsz, seq):
    rpw = bsz // NW
    splits = [(0, 104), (104, seq - 104)]
    npw = rpw * seq
    ngroups = rpw // NBUF
    mesh = plsc.VectorSubcoreMesh(core_axis_name="c", subcore_axis_name="s")

    @functools.partial(
        pl.kernel,
        mesh=mesh,
        out_type=jax.ShapeDtypeStruct((bsz * seq, 2 * EMB), jnp.float32),
        scratch_types=[
            pltpu.VMEM((npw + LANES,), jnp.int32),
            pltpu.VMEM((npw + LANES,), jnp.int32),
            pltpu.VMEM((NBUF, seq, EMB), jnp.float32),
            pltpu.VMEM((MAXLEN, EMB), jnp.float32),
            pltpu.SemaphoreType.DMA((NBUF,)),
            pltpu.SemaphoreType.DMA((NBUF,)),
        ],
        compiler_params=pltpu.CompilerParams(
            use_tc_tiling_on_sc=False, needs_layout_passes=False),
    )
    def k(x_hbm, p_hbm, table_hbm, penc_hbm, out_hbm,
          xi_v, pi_v, rows_v, p_v, sem_t, sem_s):
        cid = lax.axis_index("c")
        sid = lax.axis_index("s")
        wid = sid * NC + cid
        row0 = wid * rpw
        base = wid * npw

        pltpu.sync_copy(penc_hbm, p_v)
        pltpu.sync_copy(x_hbm.at[pl.ds(base, npw)], xi_v.at[pl.ds(0, npw)])
        pltpu.sync_copy(p_hbm.at[pl.ds(base, npw)], pi_v.at[pl.ds(0, npw)])

        lane = lax.iota(jnp.int32, LANES)

        def issue_gather(i, b):
            for off, ln in splits:
                xi = xi_v.at[pl.ds(i * seq + off, ln)]
                pltpu.async_copy(
                    table_hbm.at[xi], rows_v.at[b, pl.ds(off, ln)],
                    sem_t.at[b])

        def wait_gather(i, b):
            for off, ln in splits:
                xi = xi_v.at[pl.ds(i * seq + off, ln)]
                pltpu.make_async_copy(
                    table_hbm.at[xi], rows_v.at[b, pl.ds(off, ln)],
                    sem_t.at[b]).wait()

        def issue_store(i, b):
            pltpu.async_copy(
                rows_v.at[b],
                out_hbm.at[pl.ds((row0 + i) * seq, seq), pl.ds(0, EMB)],
                sem_s.at[b])

        def wait_store(b):
            pltpu.make_async_copy(
                rows_v.at[b],
                out_hbm.at[pl.ds(row0 * seq, seq), pl.ds(0, EMB)],
                sem_s.at[b]).wait()

        for b in range(NBUF - 1):
            issue_gather(b, b)

        nfull = seq // LANES
        tail = seq - nfull * LANES

        def group(gr, carry):
            for b in range(NBUF):
                i = gr * NBUF + b
                pb = (b + NBUF - 1) % NBUF
                pi_row = i + NBUF - 1
                if b == 0:
                    @pl.when(gr >= 1)
                    def _():
                        wait_store(pb)

                    issue_gather(pi_row, pb)
                else:
                    @pl.when(pi_row < rpw)
                    def _():
                        wait_store(pb)
                        issue_gather(pi_row, pb)

                wait_gather(i, b)

                def add16(off, nj):
                    pos16 = pi_v[pl.ds(i * seq + off, LANES)]
                    for j in range(nj):
                        r = off + j
                        prow = jnp.full((LANES,), pos16[j], dtype=jnp.int32)
                        for e in range(EMB // LANES):
                            sl = pl.ds(e * LANES, LANES)
                            pe = plsc.load_gather(p_v, [prow, lane + (e * LANES)])
                            rows_v[b, r, sl] = rows_v[b, r, sl] + pe

                def addrows(rr, c):
                    add16(rr * LANES, LANES)
                    return c

                lax.fori_loop(0, nfull, addrows, 0)
                if tail:
                    add16(nfull * LANES, tail)

                issue_store(i, b)
            return carry

        lax.fori_loop(0, ngroups, group, 0)

        for j in range(NBUF):
            wait_store((rpw - NBUF + j) % NBUF)

    return k(xf, pf, table, penc)


def kernel(x, pos, table):
    B, L = x.shape
    n = B * L
    xf = x.reshape(n).astype(jnp.int32)
    pf = pos.reshape(n).astype(jnp.int32)
    penc = _sincos_position_encoding(MAXLEN, EMB)
    tbl_compact = _tc_repack(jnp.transpose(table)).reshape(VOCAB, EMB)
    out = _sc_lookup(xf, pf, tbl_compact, penc, B, L)
    return out[:, :EMB].reshape(B, L, EMB)


# speedup vs baseline: 1.1602x; 1.1602x over previous
import functools

import jax
import jax.numpy as jnp
from jax import lax
from jax.experimental import pallas as pl
from jax.experimental.pallas import tpu as pltpu
from jax.experimental.pallas import tpu_sc as plsc

VOCAB = 1000000
EMB = 64
MAXLEN = 200

NC = 2
NS = 16
NW = NC * NS
LANES = 16
NBUF = 4

TCW = 8192


def _sincos_position_encoding(max_length, embedding_dim, n=10000):
    k = jnp.arange(max_length, dtype=jnp.float32)[:, None]
    i = jnp.arange(embedding_dim // 2, dtype=jnp.float32)[None, :]
    denominator = jnp.power(float(n), 2.0 * i / embedding_dim)
    P = jnp.zeros((max_length, embedding_dim), dtype=jnp.float32)
    P = P.at[:, 0::2].set(jnp.sin(k / denominator))
    P = P.at[:, 1::2].set(jnp.cos(k / denominator))
    return P


def _tc_repack(table_t):
    nrows, ncols = table_t.shape

    def body(in_ref, out_ref):
        x = in_ref[...]
        t = jnp.transpose(x)
        t3 = t.reshape(TCW // 2, 2, EMB)
        out_ref[:, 0:EMB] = t3[:, 0, :]
        out_ref[:, EMB:2 * EMB] = t3[:, 1, :]

    grid = (ncols + TCW - 1) // TCW
    return pl.pallas_call(
        body,
        grid=(grid,),
        in_specs=[pl.BlockSpec((nrows, TCW), lambda i: (0, i))],
        out_specs=pl.BlockSpec((TCW // 2, 2 * EMB), lambda i: (i, 0)),
        out_shape=jax.ShapeDtypeStruct((ncols // 2, 2 * EMB), jnp.float32),
    )(table_t)


@functools.partial(jax.jit, static_argnames=("bsz", "seq"))
def _sc_lookup(xf, pf, table, penc, bsz, seq):
    rpw = bsz // NW
    splits = [(0, 104), (104, seq - 104)]
    npw = rpw * seq
    ngroups = rpw // NBUF
    mesh = plsc.VectorSubcoreMesh(core_axis_name="c", subcore_axis_name="s")

    @functools.partial(
        pl.kernel,
        mesh=mesh,
        out_type=jax.ShapeDtypeStruct((bsz * seq, 2 * EMB), jnp.float32),
        scratch_types=[
            pltpu.VMEM((npw + LANES,), jnp.int32),
            pltpu.VMEM((npw + LANES,), jnp.int32),
            pltpu.VMEM((NBUF, seq, EMB), jnp.float32),
            pltpu.VMEM((MAXLEN, EMB), jnp.float32),
            pltpu.SemaphoreType.DMA((NBUF,)),
            pltpu.SemaphoreType.DMA((NBUF,)),
        ],
        compiler_params=pltpu.CompilerParams(
            use_tc_tiling_on_sc=False, needs_layout_passes=False),
    )
    def k(x_hbm, p_hbm, table_hbm, penc_hbm, out_hbm,
          xi_v, pi_v, rows_v, p_v, sem_t, sem_s):
        cid = lax.axis_index("c")
        sid = lax.axis_index("s")
        wid = sid * NC + cid
        row0 = wid * rpw
        base = wid * npw

        pltpu.sync_copy(penc_hbm, p_v)
        pltpu.sync_copy(x_hbm.at[pl.ds(base, npw)], xi_v.at[pl.ds(0, npw)])
        pltpu.sync_copy(p_hbm.at[pl.ds(base, npw)], pi_v.at[pl.ds(0, npw)])

        lane = lax.iota(jnp.int32, LANES)

        def issue_gather(i, b):
            for off, ln in splits:
                xi = xi_v.at[pl.ds(i * seq + off, ln)]
                pltpu.async_copy(
                    table_hbm.at[xi], rows_v.at[b, pl.ds(off, ln)],
                    sem_t.at[b])

        def wait_gather(i, b):
            for off, ln in splits:
                xi = xi_v.at[pl.ds(i * seq + off, ln)]
                pltpu.make_async_copy(
                    table_hbm.at[xi], rows_v.at[b, pl.ds(off, ln)],
                    sem_t.at[b]).wait()

        def issue_store(i, b):
            pltpu.async_copy(
                rows_v.at[b],
                out_hbm.at[pl.ds((row0 + i) * seq, seq), pl.ds(0, EMB)],
                sem_s.at[b])

        def wait_store(b):
            pltpu.make_async_copy(
                rows_v.at[b],
                out_hbm.at[pl.ds(row0 * seq, seq), pl.ds(0, EMB)],
                sem_s.at[b]).wait()

        for b in range(NBUF - 1):
            issue_gather(b, b)

        nfull = seq // LANES
        tail = seq - nfull * LANES

        def group(gr, carry):
            for b in range(NBUF):
                i = gr * NBUF + b
                pb = (b + NBUF - 1) % NBUF
                pi_row = i + NBUF - 1
                if b == 0:
                    @pl.when(gr >= 1)
                    def _():
                        wait_store(pb)

                    issue_gather(pi_row, pb)
                else:
                    @pl.when(pi_row < rpw)
                    def _():
                        wait_store(pb)
                        issue_gather(pi_row, pb)

                wait_gather(i, b)

                def add16(off, nj):
                    pos16 = pi_v[pl.ds(i * seq + off, LANES)]
                    for j in range(nj):
                        r = off + j
                        prow = jnp.full((LANES,), pos16[j], dtype=jnp.int32)
                        for e in range(EMB // LANES):
                            sl = pl.ds(e * LANES, LANES)
                            pe = plsc.load_gather(p_v, [prow, lane + (e * LANES)])
                            rows_v[b, r, sl] = rows_v[b, r, sl] + pe

                def addrows(rr, c):
                    add16(rr * LANES, LANES)
                    return c

                lax.fori_loop(0, nfull, addrows, 0)
                if tail:
                    add16(nfull * LANES, tail)

                issue_store(i, b)
            return carry

        lax.fori_loop(0, ngroups, group, 0)

        for j in range(NBUF):
            wait_store((rpw - NBUF + j) % NBUF)

    return k(xf, pf, table, penc)


def kernel(x, pos, table):
    B, L = x.shape
    n = B * L
    xf = x.reshape(n).astype(jnp.int32)
    pf = pos.reshape(n).astype(jnp.int32)
    penc = _sincos_position_encoding(MAXLEN, EMB)
    tbl_compact = _tc_repack(jnp.transpose(table)).reshape(VOCAB, EMB)
    out = _sc_lookup(xf, pf, tbl_compact, penc, B, L)
    return out[:, :EMB].reshape(B, L, EMB)


# speedup vs baseline: 1.1695x; 1.0081x over previous
import functools

import jax
import jax.numpy as jnp
from jax import lax
from jax.experimental import pallas as pl
from jax.experimental.pallas import tpu as pltpu
from jax.experimental.pallas import tpu_sc as plsc

VOCAB = 1000000
EMB = 64
MAXLEN = 200

NC = 2
NS = 16
NW = NC * NS
LANES = 16
NBUF = 4

TCW = 16384


def _sincos_position_encoding(max_length, embedding_dim, n=10000):
    k = jnp.arange(max_length, dtype=jnp.float32)[:, None]
    i = jnp.arange(embedding_dim // 2, dtype=jnp.float32)[None, :]
    denominator = jnp.power(float(n), 2.0 * i / embedding_dim)
    P = jnp.zeros((max_length, embedding_dim), dtype=jnp.float32)
    P = P.at[:, 0::2].set(jnp.sin(k / denominator))
    P = P.at[:, 1::2].set(jnp.cos(k / denominator))
    return P


def _tc_repack(table_t):
    nrows, ncols = table_t.shape

    def body(in_ref, out_ref):
        x = in_ref[...]
        t = jnp.transpose(x)
        t3 = t.reshape(TCW // 2, 2, EMB)
        out_ref[:, 0:EMB] = t3[:, 0, :]
        out_ref[:, EMB:2 * EMB] = t3[:, 1, :]

    grid = (ncols + TCW - 1) // TCW
    return pl.pallas_call(
        body,
        grid=(grid,),
        in_specs=[pl.BlockSpec((nrows, TCW), lambda i: (0, i))],
        out_specs=pl.BlockSpec((TCW // 2, 2 * EMB), lambda i: (i, 0)),
        out_shape=jax.ShapeDtypeStruct((ncols // 2, 2 * EMB), jnp.float32),
    )(table_t)


@functools.partial(jax.jit, static_argnames=("bsz", "seq"))
def _sc_lookup(xf, pf, table, penc, bsz, seq):
    rpw = bsz // NW
    splits = [(0, 104), (104, seq - 104)]
    npw = rpw * seq
    ngroups = rpw // NBUF
    mesh = plsc.VectorSubcoreMesh(core_axis_name="c", subcore_axis_name="s")

    @functools.partial(
        pl.kernel,
        mesh=mesh,
        out_type=jax.ShapeDtypeStruct((bsz * seq, 2 * EMB), jnp.float32),
        scratch_types=[
            pltpu.VMEM((npw + LANES,), jnp.int32),
            pltpu.VMEM((npw + LANES,), jnp.int32),
            pltpu.VMEM((NBUF, seq, EMB), jnp.float32),
            pltpu.VMEM((MAXLEN, EMB), jnp.float32),
            pltpu.SemaphoreType.DMA((NBUF,)),
            pltpu.SemaphoreType.DMA((NBUF,)),
        ],
        compiler_params=pltpu.CompilerParams(
            use_tc_tiling_on_sc=False, needs_layout_passes=False),
    )
    def k(x_hbm, p_hbm, table_hbm, penc_hbm, out_hbm,
          xi_v, pi_v, rows_v, p_v, sem_t, sem_s):
        cid = lax.axis_index("c")
        sid = lax.axis_index("s")
        wid = sid * NC + cid
        row0 = wid * rpw
        base = wid * npw

        pltpu.sync_copy(penc_hbm, p_v)
        pltpu.sync_copy(x_hbm.at[pl.ds(base, npw)], xi_v.at[pl.ds(0, npw)])
        pltpu.sync_copy(p_hbm.at[pl.ds(base, npw)], pi_v.at[pl.ds(0, npw)])

        lane = lax.iota(jnp.int32, LANES)

        def issue_gather(i, b):
            for off, ln in splits:
                xi = xi_v.at[pl.ds(i * seq + off, ln)]
                pltpu.async_copy(
                    table_hbm.at[xi], rows_v.at[b, pl.ds(off, ln)],
                    sem_t.at[b])

        def wait_gather(i, b):
            for off, ln in splits:
                xi = xi_v.at[pl.ds(i * seq + off, ln)]
                pltpu.make_async_copy(
                    table_hbm.at[xi], rows_v.at[b, pl.ds(off, ln)],
                    sem_t.at[b]).wait()

        def issue_store(i, b):
            pltpu.async_copy(
                rows_v.at[b],
                out_hbm.at[pl.ds((row0 + i) * seq, seq), pl.ds(0, EMB)],
                sem_s.at[b])

        def wait_store(b):
            pltpu.make_async_copy(
                rows_v.at[b],
                out_hbm.at[pl.ds(row0 * seq, seq), pl.ds(0, EMB)],
                sem_s.at[b]).wait()

        for b in range(NBUF - 1):
            issue_gather(b, b)

        nfull = seq // LANES
        tail = seq - nfull * LANES

        def group(gr, carry):
            for b in range(NBUF):
                i = gr * NBUF + b
                pb = (b + NBUF - 1) % NBUF
                pi_row = i + NBUF - 1
                if b == 0:
                    @pl.when(gr >= 1)
                    def _():
                        wait_store(pb)

                    issue_gather(pi_row, pb)
                else:
                    @pl.when(pi_row < rpw)
                    def _():
                        wait_store(pb)
                        issue_gather(pi_row, pb)

                wait_gather(i, b)

                def add16(off, nj):
                    pos16 = pi_v[pl.ds(i * seq + off, LANES)]
                    for j in range(nj):
                        r = off + j
                        prow = jnp.full((LANES,), pos16[j], dtype=jnp.int32)
                        for e in range(EMB // LANES):
                            sl = pl.ds(e * LANES, LANES)
                            pe = plsc.load_gather(p_v, [prow, lane + (e * LANES)])
                            rows_v[b, r, sl] = rows_v[b, r, sl] + pe

                def addrows(rr, c):
                    add16(rr * LANES, LANES)
                    return c

                lax.fori_loop(0, nfull, addrows, 0)
                if tail:
                    add16(nfull * LANES, tail)

                issue_store(i, b)
            return carry

        lax.fori_loop(0, ngroups, group, 0)

        for j in range(NBUF):
            wait_store((rpw - NBUF + j) % NBUF)

    return k(xf, pf, table, penc)


def kernel(x, pos, table):
    B, L = x.shape
    n = B * L
    xf = x.reshape(n).astype(jnp.int32)
    pf = pos.reshape(n).astype(jnp.int32)
    penc = _sincos_position_encoding(MAXLEN, EMB)
    tbl_compact = _tc_repack(jnp.transpose(table)).reshape(VOCAB, EMB)
    out = _sc_lookup(xf, pf, tbl_compact, penc, B, L)
    return out[:, :EMB].reshape(B, L, EMB)
